# Initial kernel scaffold; baseline (speedup 1.0000x reference)
import functools

import jax
import jax.numpy as jnp
from jax import lax
from jax.experimental import pallas as pl
from jax.experimental.pallas import tpu as pltpu
from jax.experimental.pallas import tpu_sc as plsc

N_NODES = 10000
N_CTX = 10000
N_EDGES = 320000
D = 128

N_TILES = 16
CHUNK = 128
STEPS = 157
EDGES_PER_TILE = CHUNK * STEPS
E_PAD = EDGES_PER_TILE * N_TILES
ACC_ROWS = 10240
ROWS_PER_TILE = ACC_ROWS // N_TILES
CNT_W = 16


def _sc_segment_sums(xA, srcA, dstA, xB, srcB, dstB, zrow, zcnt, ones):
    mesh = plsc.VectorSubcoreMesh(core_axis_name="c", subcore_axis_name="s")

    @functools.partial(
        pl.kernel,
        mesh=mesh,
        out_type=[
            jax.ShapeDtypeStruct((ACC_ROWS, D), jnp.float32),
            jax.ShapeDtypeStruct((ACC_ROWS, CNT_W), jnp.float32),
            jax.ShapeDtypeStruct((ACC_ROWS, D), jnp.float32),
            jax.ShapeDtypeStruct((ACC_ROWS, CNT_W), jnp.float32),
        ],
        scratch_types=[
            pltpu.VMEM((STEPS, CHUNK), jnp.int32),
            pltpu.VMEM((STEPS, CHUNK), jnp.int32),
            pltpu.VMEM((CHUNK, D), jnp.float32),
            pltpu.VMEM((CHUNK, CNT_W), jnp.float32),
            pltpu.VMEM((ROWS_PER_TILE, CNT_W), jnp.float32),
            pltpu.VMEM_SHARED((ACC_ROWS, D), jnp.float32),
            pltpu.VMEM_SHARED((ACC_ROWS, CNT_W), jnp.float32),
            pltpu.SemaphoreType.DMA,
        ],
    )
    def k(xA_h, srcA_h, dstA_h, xB_h, srcB_h, dstB_h, zrow_h, zcnt_h, ones_h,
          sumA_h, cntA_h, sumB_h, cntB_h,
          src_t, dst_t, rows_t, ones_t, zc_t, acc_s, cnt_s, sem):
        c = lax.axis_index("c")
        s = lax.axis_index("s")
        row0 = s * ROWS_PER_TILE

        pltpu.sync_copy(zrow_h, rows_t)
        pltpu.sync_copy(zcnt_h, zc_t)
        for kk in range(ROWS_PER_TILE // CHUNK):
            pltpu.sync_copy(rows_t, acc_s.at[pl.ds(row0 + kk * CHUNK, CHUNK)])
        pltpu.sync_copy(zc_t, cnt_s.at[pl.ds(row0, ROWS_PER_TILE)])
        pltpu.sync_copy(ones_h, ones_t)
        plsc.subcore_barrier()

        def run(x_h, src_h, dst_h, sum_h, cnt_h):
            pltpu.sync_copy(src_h.at[pl.ds(s * STEPS, STEPS)], src_t)
            pltpu.sync_copy(dst_h.at[pl.ds(s * STEPS, STEPS)], dst_t)

            def body(j, carry):
                pltpu.async_copy(x_h.at[src_t.at[j]], rows_t, sem).wait()
                pltpu.sync_copy(rows_t, acc_s.at[dst_t.at[j]], add=True)
                pltpu.sync_copy(ones_t, cnt_s.at[dst_t.at[j]], add=True)
                return carry

            lax.fori_loop(0, STEPS, body, 0)
            plsc.subcore_barrier()

            for kk in range(ROWS_PER_TILE // CHUNK):
                r = row0 + kk * CHUNK
                pltpu.sync_copy(acc_s.at[pl.ds(r, CHUNK)], rows_t)
                pltpu.sync_copy(rows_t, sum_h.at[pl.ds(r, CHUNK)])
            pltpu.sync_copy(cnt_s.at[pl.ds(row0, ROWS_PER_TILE)], zc_t)
            pltpu.sync_copy(zc_t, cnt_h.at[pl.ds(row0, ROWS_PER_TILE)])

        @pl.when(c == 0)
        def _():
            run(xA_h, srcA_h, dstA_h, sumA_h, cntA_h)

        @pl.when(c == 1)
        def _():
            run(xB_h, srcB_h, dstB_h, sumB_h, cntB_h)

    return k(xA, srcA, dstA, xB, srcB, dstB, zrow, zcnt, ones)


def _tc_body(sumA, cntA, sumB, cntB, ctx, wsrc, wdst2, b2, out):
    cA = jnp.maximum(cntA[:, 0:1], 1.0)
    cB = jnp.maximum(cntB[:, 0:1], 1.0)
    m = sumA[...] / cA + sumB[...] / cB
    acc = jnp.dot(m, wsrc[...], preferred_element_type=jnp.float32)
    acc += jnp.dot(ctx[...], wdst2[...], preferred_element_type=jnp.float32)
    out[...] = acc + b2[...]


def _tc_epilogue(sumA, cntA, sumB, cntB, context, W_src, W_dst2, b2):
    blk = 1000
    grid = (N_CTX // blk,)
    return pl.pallas_call(
        _tc_body,
        grid=grid,
        in_specs=[
            pl.BlockSpec((blk, D), lambda i: (i, 0)),
            pl.BlockSpec((blk, CNT_W), lambda i: (i, 0)),
            pl.BlockSpec((blk, D), lambda i: (i, 0)),
            pl.BlockSpec((blk, CNT_W), lambda i: (i, 0)),
            pl.BlockSpec((blk, D), lambda i: (i, 0)),
            pl.BlockSpec((D, D), lambda i: (0, 0)),
            pl.BlockSpec((D, D), lambda i: (0, 0)),
            pl.BlockSpec((1, D), lambda i: (0, 0)),
        ],
        out_specs=pl.BlockSpec((blk, D), lambda i: (i, 0)),
        out_shape=jax.ShapeDtypeStruct((N_CTX, D), jnp.float32),
    )(sumA, cntA, sumB, cntB, context, W_src, W_dst2, b2)


def _prep_edges(edges):
    src = edges[0].astype(jnp.int32)
    dst = edges[1].astype(jnp.int32)
    pad = E_PAD - N_EDGES
    src = jnp.concatenate([src, jnp.zeros((pad,), jnp.int32)])
    dst = jnp.concatenate([dst, jnp.full((pad,), N_CTX, jnp.int32)])
    return (src.reshape(N_TILES * STEPS, CHUNK),
            dst.reshape(N_TILES * STEPS, CHUNK))


def kernel(xA, edgesA, xB, edgesB, context, W_src, W_dst, b):
    srcA, dstA = _prep_edges(edgesA)
    srcB, dstB = _prep_edges(edgesB)
    zrow = jnp.zeros((CHUNK, D), jnp.float32)
    zcnt = jnp.zeros((ROWS_PER_TILE, CNT_W), jnp.float32)
    ones = jnp.ones((CHUNK, CNT_W), jnp.float32)
    sumA, cntA, sumB, cntB = _sc_segment_sums(
        xA, srcA, dstA, xB, srcB, dstB, zrow, zcnt, ones)
    sumA = sumA[:N_CTX]
    cntA = cntA[:N_CTX]
    sumB = sumB[:N_CTX]
    cntB = cntB[:N_CTX]
    return _tc_epilogue(sumA, cntA, sumB, cntB, context,
                        W_src, 2.0 * W_dst, (2.0 * b).reshape(1, D))


# speedup vs baseline: 5.0222x; 5.0222x over previous
import functools

import jax
import jax.numpy as jnp
from jax import lax
from jax.experimental import pallas as pl
from jax.experimental.pallas import tpu as pltpu
from jax.experimental.pallas import tpu_sc as plsc

N_NODES = 10000
N_CTX = 10000
N_EDGES = 320000
D = 128
L = 16

N_TILES = 16
CHUNK = 128
IDX_BLK = 8
N_BLKS = 20
STEPS = IDX_BLK * N_BLKS
EDGES_PER_TILE = CHUNK * STEPS
E_PAD = EDGES_PER_TILE * N_TILES
IDX_ROWS = E_PAD // CHUNK
ACC_ROWS = 10112
ROWS_PER_TILE = ACC_ROWS // N_TILES
FULL = ROWS_PER_TILE // CHUNK
REM = ROWS_PER_TILE % CHUNK
HIST = 10240


_GATHER_DNUMS = lax.GatherDimensionNumbers(
    offset_dims=(), collapsed_slice_dims=(0,), start_index_map=(0,))


def _vgather(x, idx):
    return lax.gather(x, idx[:, None], _GATHER_DNUMS, slice_sizes=(1,),
                      mode=lax.GatherScatterMode.PROMISE_IN_BOUNDS)


def _rot_tables():
    iota = lax.iota(jnp.int32, L)
    rots = [(iota + k) & (L - 1) for k in range(1, L)]
    jlt = [r < iota for r in rots]
    return rots, jlt


def _vreg_counts(d, rots, jlt):
    cnt = jnp.ones((L,), jnp.float32)
    viol = jnp.zeros((L,), jnp.bool_)
    for k in range(L - 1):
        eq = d == _vgather(d, rots[k])
        cnt = cnt + jnp.where(eq, 1.0, 0.0)
        viol = viol | (eq & jlt[k])
    return cnt, jnp.logical_not(viol)


def _sc_segment_sums(x2, src2, dst2, zrow):
    mesh = plsc.VectorSubcoreMesh(core_axis_name="c", subcore_axis_name="s")

    @functools.partial(
        pl.kernel,
        mesh=mesh,
        compiler_params=pltpu.CompilerParams(needs_layout_passes=False),
        out_type=[
            jax.ShapeDtypeStruct((2 * ACC_ROWS, D), jnp.float32),
            jax.ShapeDtypeStruct((2 * N_TILES, HIST), jnp.float32),
        ],
        scratch_types=[
            pltpu.VMEM((IDX_BLK, CHUNK), jnp.int32),
            pltpu.VMEM((IDX_BLK, CHUNK), jnp.int32),
            pltpu.VMEM((CHUNK, D), jnp.float32),
            pltpu.VMEM((HIST,), jnp.float32),
            pltpu.VMEM_SHARED((ACC_ROWS, D), jnp.float32),
            pltpu.SemaphoreType.DMA,
        ],
    )
    def k(x_h, src_h, dst_h, zrow_h, sum_h, cnt_h,
          src_t, dst_t, rows_t, hist_t, acc_s, sem):
        g = lax.axis_index("c")
        s = lax.axis_index("s")
        row0 = s * ROWS_PER_TILE

        pltpu.sync_copy(zrow_h, rows_t)
        for kk in range(FULL):
            pltpu.sync_copy(rows_t, acc_s.at[pl.ds(row0 + kk * CHUNK, CHUNK)])
        pltpu.sync_copy(rows_t.at[pl.ds(0, REM)],
                        acc_s.at[pl.ds(row0 + FULL * CHUNK, REM)])

        zv = jnp.zeros((L,), jnp.float32)

        def zero_hist(i, carry):
            hist_t[pl.ds(i * L, L)] = zv
            return carry

        lax.fori_loop(0, HIST // L, zero_hist, 0)
        plsc.subcore_barrier()

        rots, jlt = _rot_tables()

        def outer(blk, carry):
            base = g * IDX_ROWS + s * STEPS + blk * IDX_BLK
            pltpu.sync_copy(src_h.at[pl.ds(base, IDX_BLK)], src_t)
            pltpu.sync_copy(dst_h.at[pl.ds(base, IDX_BLK)], dst_t)

            def body(j, carry2):
                cp = pltpu.async_copy(x_h.at[src_t.at[j]], rows_t, sem)
                for lane in range(CHUNK // L):
                    d = dst_t[j, pl.ds(lane * L, L)]
                    cnt, first = _vreg_counts(d, rots, jlt)
                    plsc.addupdate_scatter(hist_t, [d], cnt, mask=first)
                cp.wait()
                pltpu.sync_copy(rows_t, acc_s.at[dst_t.at[j]], add=True)
                return carry2

            return lax.fori_loop(0, IDX_BLK, body, carry)

        lax.fori_loop(0, N_BLKS, outer, 0)
        plsc.subcore_barrier()

        out0 = g * ACC_ROWS + row0
        for kk in range(FULL):
            pltpu.sync_copy(acc_s.at[pl.ds(row0 + kk * CHUNK, CHUNK)], rows_t)
            pltpu.sync_copy(rows_t, sum_h.at[pl.ds(out0 + kk * CHUNK, CHUNK)])
        pltpu.sync_copy(acc_s.at[pl.ds(row0 + FULL * CHUNK, REM)],
                        rows_t.at[pl.ds(0, REM)])
        pltpu.sync_copy(rows_t.at[pl.ds(0, REM)],
                        sum_h.at[pl.ds(out0 + FULL * CHUNK, REM)])
        pltpu.sync_copy(hist_t, cnt_h.at[g * N_TILES + s])

    return k(x2, src2, dst2, zrow)


def _tc_body(sumA, cntA, sumB, cntB, ctx, wsrc, wdst2, b2, out):
    cA = jnp.maximum(jnp.sum(cntA[...], axis=1, keepdims=True), 1.0)
    cB = jnp.maximum(jnp.sum(cntB[...], axis=1, keepdims=True), 1.0)
    m = sumA[...] / cA + sumB[...] / cB
    acc = jnp.dot(m, wsrc[...], preferred_element_type=jnp.float32)
    acc += jnp.dot(ctx[...], wdst2[...], preferred_element_type=jnp.float32)
    out[...] = acc + b2[...]


def _tc_epilogue(sumA, cntA, sumB, cntB, context, W_src, W_dst2, b2):
    blk = 1000
    grid = (N_CTX // blk,)
    return pl.pallas_call(
        _tc_body,
        grid=grid,
        in_specs=[
            pl.BlockSpec((blk, D), lambda i: (i, 0)),
            pl.BlockSpec((blk, N_TILES), lambda i: (i, 0)),
            pl.BlockSpec((blk, D), lambda i: (i, 0)),
            pl.BlockSpec((blk, N_TILES), lambda i: (i, 0)),
            pl.BlockSpec((blk, D), lambda i: (i, 0)),
            pl.BlockSpec((D, D), lambda i: (0, 0)),
            pl.BlockSpec((D, D), lambda i: (0, 0)),
            pl.BlockSpec((1, D), lambda i: (0, 0)),
        ],
        out_specs=pl.BlockSpec((blk, D), lambda i: (i, 0)),
        out_shape=jax.ShapeDtypeStruct((N_CTX, D), jnp.float32),
    )(sumA, cntA, sumB, cntB, context, W_src, W_dst2, b2)


def _prep_edges(edges, src_off):
    src = edges[0].astype(jnp.int32) + src_off
    dst = edges[1].astype(jnp.int32)
    pad = E_PAD - N_EDGES
    src = jnp.concatenate([src, jnp.full((pad,), src_off, jnp.int32)])
    dst = jnp.concatenate([dst, jnp.full((pad,), N_CTX, jnp.int32)])
    return src.reshape(IDX_ROWS, CHUNK), dst.reshape(IDX_ROWS, CHUNK)


def kernel(xA, edgesA, xB, edgesB, context, W_src, W_dst, b):
    srcA, dstA = _prep_edges(edgesA, 0)
    srcB, dstB = _prep_edges(edgesB, N_NODES)
    x2 = jnp.concatenate([xA, xB])
    src2 = jnp.concatenate([srcA, srcB])
    dst2 = jnp.concatenate([dstA, dstB])
    zrow = jnp.zeros((CHUNK, D), jnp.float32)
    sums, hists = _sc_segment_sums(x2, src2, dst2, zrow)
    cntA = hists[:N_TILES, :N_CTX].T
    cntB = hists[N_TILES:, :N_CTX].T
    return _tc_epilogue(sums[:N_CTX], cntA,
                        sums[ACC_ROWS:ACC_ROWS + N_CTX], cntB, context,
                        W_src, 2.0 * W_dst, (2.0 * b).reshape(1, D))


# speedup vs baseline: 6.1427x; 1.2231x over previous
import functools

import jax
import jax.numpy as jnp
from jax import lax
from jax.experimental import pallas as pl
from jax.experimental.pallas import tpu as pltpu
from jax.experimental.pallas import tpu_sc as plsc

N_NODES = 10000
N_CTX = 10000
N_EDGES = 320000
D = 128
L = 16

N_TILES = 16
CHUNK = 128
IDX_BLK = 16
N_BLKS = 10
STEPS = IDX_BLK * N_BLKS
EDGES_PER_TILE = CHUNK * STEPS
E_PAD = EDGES_PER_TILE * N_TILES
IDX_ROWS = E_PAD // CHUNK
ACC_ROWS = 10112
ROWS_PER_TILE = ACC_ROWS // N_TILES
FULL = ROWS_PER_TILE // CHUNK
REM = ROWS_PER_TILE % CHUNK
HIST = 10240

_GATHER_DNUMS = lax.GatherDimensionNumbers(
    offset_dims=(), collapsed_slice_dims=(0,), start_index_map=(0,))


def _vgather(x, idx):
    return lax.gather(x, idx[:, None], _GATHER_DNUMS, slice_sizes=(1,),
                      mode=lax.GatherScatterMode.PROMISE_IN_BOUNDS)


def _rot_tables():
    iota = lax.iota(jnp.int32, L)
    rots = [(iota + k) & (L - 1) for k in range(1, L)]
    jlt = [r < iota for r in rots]
    return rots, jlt


def _vreg_counts(d, rots, jlt):
    cnt = jnp.ones((L,), jnp.float32)
    viol = jnp.zeros((L,), jnp.bool_)
    for k in range(L - 1):
        eq = d == _vgather(d, rots[k])
        cnt = cnt + jnp.where(eq, 1.0, 0.0)
        viol = viol | (eq & jlt[k])
    return cnt, jnp.logical_not(viol)


def _sc_segment_sums(x2, src2, dst2, zrow):
    mesh = plsc.VectorSubcoreMesh(core_axis_name="c", subcore_axis_name="s")

    @functools.partial(
        pl.kernel,
        mesh=mesh,
        compiler_params=pltpu.CompilerParams(needs_layout_passes=False),
        out_type=[
            jax.ShapeDtypeStruct((2 * ACC_ROWS, D), jnp.float32),
            jax.ShapeDtypeStruct((2 * N_TILES, HIST), jnp.float32),
        ],
        scratch_types=[
            pltpu.VMEM((IDX_BLK, CHUNK), jnp.int32),
            pltpu.VMEM((IDX_BLK, CHUNK), jnp.int32),
            pltpu.VMEM((CHUNK, D), jnp.float32),
            pltpu.VMEM((CHUNK, D), jnp.float32),
            pltpu.VMEM((HIST,), jnp.float32),
            pltpu.VMEM_SHARED((ACC_ROWS, D), jnp.float32),
            pltpu.SemaphoreType.DMA,
            pltpu.SemaphoreType.DMA,
        ],
    )
    def k(x_h, src_h, dst_h, zrow_h, sum_h, cnt_h,
          src_t, dst_t, rows0, rows1, hist_t, acc_s, sem0, sem1):
        g = lax.axis_index("c")
        s = lax.axis_index("s")
        row0 = s * ROWS_PER_TILE

        pltpu.sync_copy(zrow_h, rows0)
        for kk in range(FULL):
            pltpu.sync_copy(rows0, acc_s.at[pl.ds(row0 + kk * CHUNK, CHUNK)])
        pltpu.sync_copy(rows0.at[pl.ds(0, REM)],
                        acc_s.at[pl.ds(row0 + FULL * CHUNK, REM)])

        zv = jnp.zeros((L,), jnp.float32)

        def zero_hist(i, carry):
            hist_t[pl.ds(i * L, L)] = zv
            return carry

        lax.fori_loop(0, HIST // L, zero_hist, 0)
        plsc.subcore_barrier()

        rots, jlt = _rot_tables()

        def count_row(j):
            def one(lane, carry):
                d = dst_t[j, pl.ds(lane * L, L)]
                cnt, first = _vreg_counts(d, rots, jlt)
                plsc.addupdate_scatter(hist_t, [d], cnt, mask=first)
                return carry
            lax.fori_loop(0, CHUNK // L, one, 0)

        def outer(blk, carry):
            base = g * IDX_ROWS + s * STEPS + blk * IDX_BLK
            pltpu.sync_copy(src_h.at[pl.ds(base, IDX_BLK)], src_t)
            pltpu.sync_copy(dst_h.at[pl.ds(base, IDX_BLK)], dst_t)
            pltpu.async_copy(x_h.at[src_t.at[0]], rows0, sem0)

            def pair(u, carry2):
                ja = 2 * u
                jb = 2 * u + 1
                cpb = pltpu.async_copy(x_h.at[src_t.at[jb]], rows1, sem1)
                count_row(ja)
                pltpu.make_async_copy(x_h.at[src_t.at[ja]], rows0,
                                      sem0).wait()
                pltpu.sync_copy(rows0, acc_s.at[dst_t.at[ja]], add=True)

                @pl.when(u < IDX_BLK // 2 - 1)
                def _():
                    pltpu.async_copy(x_h.at[src_t.at[ja + 2]], rows0, sem0)

                count_row(jb)
                cpb.wait()
                pltpu.sync_copy(rows1, acc_s.at[dst_t.at[jb]], add=True)
                return carry2

            return lax.fori_loop(0, IDX_BLK // 2, pair, carry)

        lax.fori_loop(0, N_BLKS, outer, 0)
        plsc.subcore_barrier()

        out0 = g * ACC_ROWS + row0
        for kk in range(FULL):
            pltpu.sync_copy(acc_s.at[pl.ds(row0 + kk * CHUNK, CHUNK)], rows0)
            pltpu.sync_copy(rows0, sum_h.at[pl.ds(out0 + kk * CHUNK, CHUNK)])
        pltpu.sync_copy(acc_s.at[pl.ds(row0 + FULL * CHUNK, REM)],
                        rows0.at[pl.ds(0, REM)])
        pltpu.sync_copy(rows0.at[pl.ds(0, REM)],
                        sum_h.at[pl.ds(out0 + FULL * CHUNK, REM)])
        pltpu.sync_copy(hist_t, cnt_h.at[g * N_TILES + s])

    return k(x2, src2, dst2, zrow)


def _tc_body(sumA, cntA, sumB, cntB, ctx, wsrc, wdst2, b2, out):
    cA = jnp.maximum(jnp.sum(cntA[...], axis=1, keepdims=True), 1.0)
    cB = jnp.maximum(jnp.sum(cntB[...], axis=1, keepdims=True), 1.0)
    m = sumA[...] / cA + sumB[...] / cB
    acc = jnp.dot(m, wsrc[...], preferred_element_type=jnp.float32)
    acc += jnp.dot(ctx[...], wdst2[...], preferred_element_type=jnp.float32)
    out[...] = acc + b2[...]


def _tc_epilogue(sumA, cntA, sumB, cntB, context, W_src, W_dst2, b2):
    blk = 1000
    grid = (N_CTX // blk,)
    return pl.pallas_call(
        _tc_body,
        grid=grid,
        in_specs=[
            pl.BlockSpec((blk, D), lambda i: (i, 0)),
            pl.BlockSpec((blk, N_TILES), lambda i: (i, 0)),
            pl.BlockSpec((blk, D), lambda i: (i, 0)),
            pl.BlockSpec((blk, N_TILES), lambda i: (i, 0)),
            pl.BlockSpec((blk, D), lambda i: (i, 0)),
            pl.BlockSpec((D, D), lambda i: (0, 0)),
            pl.BlockSpec((D, D), lambda i: (0, 0)),
            pl.BlockSpec((1, D), lambda i: (0, 0)),
        ],
        out_specs=pl.BlockSpec((blk, D), lambda i: (i, 0)),
        out_shape=jax.ShapeDtypeStruct((N_CTX, D), jnp.float32),
    )(sumA, cntA, sumB, cntB, context, W_src, W_dst2, b2)


def _prep_edges(edges, src_off):
    src = edges[0].astype(jnp.int32) + src_off
    dst = edges[1].astype(jnp.int32)
    pad = E_PAD - N_EDGES
    src = jnp.concatenate([src, jnp.full((pad,), src_off, jnp.int32)])
    dst = jnp.concatenate([dst, jnp.full((pad,), N_CTX, jnp.int32)])
    return src.reshape(IDX_ROWS, CHUNK), dst.reshape(IDX_ROWS, CHUNK)


def kernel(xA, edgesA, xB, edgesB, context, W_src, W_dst, b):
    srcA, dstA = _prep_edges(edgesA, 0)
    srcB, dstB = _prep_edges(edgesB, N_NODES)
    x2 = jnp.concatenate([xA, xB])
    src2 = jnp.concatenate([srcA, srcB])
    dst2 = jnp.concatenate([dstA, dstB])
    zrow = jnp.zeros((CHUNK, D), jnp.float32)
    sums, hists = _sc_segment_sums(x2, src2, dst2, zrow)
    cntA = hists[:N_TILES, :N_CTX].T
    cntB = hists[N_TILES:, :N_CTX].T
    return _tc_epilogue(sums[:N_CTX], cntA,
                        sums[ACC_ROWS:ACC_ROWS + N_CTX], cntB, context,
                        W_src, 2.0 * W_dst, (2.0 * b).reshape(1, D))
